# baseline (speedup 1.0000x reference)
import functools

import jax
import jax.numpy as jnp
from jax import lax
from jax.experimental import pallas as pl
from jax.experimental.pallas import tpu as pltpu
from jax.experimental.pallas import tpu_sc as plsc

N = 10000
E = 100000
IN_FEATS = 128
H = 4
D = 32
HD = H * D
C = 153
NEG_SLOPE = 0.2

NP = 10240
NC = 2
NS = 16
NW = NC * NS
ET = E // NW
CH = 64
NCHUNK = (ET + CH - 1) // CH
ETP = NCHUNK * CH
RPT = NP // NS
ZR = 32

_BLK = 1024


def _tc_dense(h, W3, A3, B3):
    def body(h_ref, w_ref, a_ref, b_ref, z_ref, el_ref, er_ref):
        hb = h_ref[...]
        for r in range(3):
            z = jnp.dot(hb, w_ref[r], preferred_element_type=jnp.float32)
            z_ref[r] = z
            el_ref[r] = jnp.dot(z, a_ref[r], preferred_element_type=jnp.float32)
            er_ref[r] = jnp.dot(z, b_ref[r], preferred_element_type=jnp.float32)

    return pl.pallas_call(
        body,
        grid=(NP // _BLK,),
        in_specs=[
            pl.BlockSpec((_BLK, IN_FEATS), lambda i: (i, 0)),
            pl.BlockSpec((3, IN_FEATS, HD), lambda i: (0, 0, 0)),
            pl.BlockSpec((3, HD, 16), lambda i: (0, 0, 0)),
            pl.BlockSpec((3, HD, 16), lambda i: (0, 0, 0)),
        ],
        out_specs=[
            pl.BlockSpec((3, _BLK, HD), lambda i: (0, i, 0)),
            pl.BlockSpec((3, _BLK, 16), lambda i: (0, i, 0)),
            pl.BlockSpec((3, _BLK, 16), lambda i: (0, i, 0)),
        ],
        out_shape=[
            jax.ShapeDtypeStruct((3, NP, HD), jnp.float32),
            jax.ShapeDtypeStruct((3, NP, 16), jnp.float32),
            jax.ShapeDtypeStruct((3, NP, 16), jnp.float32),
        ],
    )(h, W3, A3, B3)


def _head_expand_mat():
    col = lax.broadcasted_iota(jnp.int32, (H, HD), 1)
    row = lax.broadcasted_iota(jnp.int32, (H, HD), 0)
    return (col // D == row).astype(jnp.float32)


def _combine(acc_refs, s_refs, bias_ref):
    emat = _head_expand_mat()
    hsum = None
    for r in range(3):
        acc = acc_refs[r][0] + acc_refs[r][1]
        s = s_refs[r][0][:, :H] + s_refs[r][1][:, :H]
        inv = 1.0 / (s + 1e-9)
        sexp = jnp.dot(inv, emat, preferred_element_type=jnp.float32)
        t = acc * sexp + bias_ref[r][None, :]
        hsum = t if hsum is None else hsum + t
    return hsum


def _tc_combine_dense(accs, ss, bias3, W3, A3, B3):
    def body(a0, a1, a2, s0, s1, s2, bias_ref, w_ref, a3_ref, b3_ref,
             z_ref, el_ref, er_ref):
        h = jnp.maximum(_combine((a0, a1, a2), (s0, s1, s2), bias_ref), 0.0)
        for r in range(3):
            z = jnp.dot(h, w_ref[r], preferred_element_type=jnp.float32)
            z_ref[r] = z
            el_ref[r] = jnp.dot(z, a3_ref[r], preferred_element_type=jnp.float32)
            er_ref[r] = jnp.dot(z, b3_ref[r], preferred_element_type=jnp.float32)

    acc_spec = pl.BlockSpec((NC, _BLK, HD), lambda i: (0, i, 0))
    s_spec = pl.BlockSpec((NC, _BLK, 16), lambda i: (0, i, 0))
    return pl.pallas_call(
        body,
        grid=(NP // _BLK,),
        in_specs=[acc_spec, acc_spec, acc_spec, s_spec, s_spec, s_spec,
                  pl.BlockSpec((3, HD), lambda i: (0, 0)),
                  pl.BlockSpec((3, IN_FEATS, HD), lambda i: (0, 0, 0)),
                  pl.BlockSpec((3, HD, 16), lambda i: (0, 0, 0)),
                  pl.BlockSpec((3, HD, 16), lambda i: (0, 0, 0))],
        out_specs=[
            pl.BlockSpec((3, _BLK, HD), lambda i: (0, i, 0)),
            pl.BlockSpec((3, _BLK, 16), lambda i: (0, i, 0)),
            pl.BlockSpec((3, _BLK, 16), lambda i: (0, i, 0)),
        ],
        out_shape=[
            jax.ShapeDtypeStruct((3, NP, HD), jnp.float32),
            jax.ShapeDtypeStruct((3, NP, 16), jnp.float32),
            jax.ShapeDtypeStruct((3, NP, 16), jnp.float32),
        ],
    )(*accs, *ss, bias3, W3, A3, B3)


def _tc_combine_linear(accs, ss, bias3, linW_pad, linb_pad):
    def body(a0, a1, a2, s0, s1, s2, bias_ref, w_ref, b_ref, out_ref):
        h = _combine((a0, a1, a2), (s0, s1, s2), bias_ref)
        out_ref[...] = (jnp.dot(h, w_ref[...], preferred_element_type=jnp.float32)
                        + b_ref[...][None, :])

    acc_spec = pl.BlockSpec((NC, _BLK, HD), lambda i: (0, i, 0))
    s_spec = pl.BlockSpec((NC, _BLK, 16), lambda i: (0, i, 0))
    return pl.pallas_call(
        body,
        grid=(NP // _BLK,),
        in_specs=[acc_spec, acc_spec, acc_spec, s_spec, s_spec, s_spec,
                  pl.BlockSpec((3, HD), lambda i: (0, 0)),
                  pl.BlockSpec((HD, 256), lambda i: (0, 0)),
                  pl.BlockSpec((256,), lambda i: (0,))],
        out_specs=pl.BlockSpec((_BLK, 256), lambda i: (i, 0)),
        out_shape=jax.ShapeDtypeStruct((NP, 256), jnp.float32),
    )(*accs, *ss, bias3, linW_pad, linb_pad)


_SPLAT_DNUMS = lax.GatherDimensionNumbers(
    offset_dims=(), collapsed_slice_dims=(0,), start_index_map=(0,))


def _splat(v, lane):
    idx = jnp.full((16, 1), lane, jnp.int32)
    return lax.gather(v, idx, _SPLAT_DNUMS, (1,),
                      mode=lax.GatherScatterMode.PROMISE_IN_BOUNDS)


def _sc_layer(zs, els, ers, src4, dst4):
    mesh = plsc.VectorSubcoreMesh(core_axis_name="c", subcore_axis_name="s")
    out_type = ([jax.ShapeDtypeStruct((NC, NP, HD), jnp.float32) for _ in range(3)]
                + [jax.ShapeDtypeStruct((NC, NP, 16), jnp.float32) for _ in range(3)])

    @functools.partial(
        pl.kernel,
        out_type=out_type,
        mesh=mesh,
        compiler_params=pltpu.CompilerParams(use_tc_tiling_on_sc=False),
        scratch_types=[
            pltpu.VMEM_SHARED((NP, HD), jnp.float32),
            pltpu.VMEM_SHARED((NP, 16), jnp.float32),
            pltpu.VMEM((NCHUNK, CH), jnp.int32),
            pltpu.VMEM((NCHUNK, CH), jnp.int32),
            [pltpu.VMEM((CH, HD), jnp.float32) for _ in range(2)],
            [pltpu.VMEM((CH, 16), jnp.float32) for _ in range(2)],
            [pltpu.VMEM((CH, 16), jnp.float32) for _ in range(2)],
            pltpu.VMEM((ZR, HD), jnp.float32),
            pltpu.VMEM((ZR, 16), jnp.float32),
            [pltpu.SemaphoreType.DMA for _ in range(3)],
            [pltpu.SemaphoreType.DMA for _ in range(3)],
            [pltpu.SemaphoreType.DMA for _ in range(2)],
        ],
    )
    def k(z0, z1, z2, el0, el1, el2, er0, er1, er2, src_h, dst_h,
          acc0, acc1, acc2, sd0, sd1, sd2,
          acc_sh, s_sh, srcv, dstv, rows2, elg2, erg2, zacc, zs_v,
          semA, semB, semZ):
        cid = lax.axis_index("c")
        sid = lax.axis_index("s")
        wid = cid * NS + sid
        r0 = sid * RPT

        zero16 = jnp.zeros((16,), jnp.float32)

        def zinit(i, _):
            for j in range(HD // 16):
                zacc[i, pl.ds(j * 16, 16)] = zero16
            zs_v[i, :] = zero16
            return _
        lax.fori_loop(0, ZR, zinit, None)

        z_in = (z0, z1, z2)
        el_in = (el0, el1, el2)
        er_in = (er0, er1, er2)
        acc_out = (acc0, acc1, acc2)
        s_out = (sd0, sd1, sd2)

        def issue(r, ch, b):
            pltpu.async_copy(z_in[r].at[srcv.at[ch]], rows2[b], semA[0] if b == 0 else semB[0])
            pltpu.async_copy(el_in[r].at[srcv.at[ch]], elg2[b], semA[1] if b == 0 else semB[1])
            pltpu.async_copy(er_in[r].at[dstv.at[ch]], erg2[b], semB[2] if b else semA[2])

        def wait(r, b):
            sems = semA if b == 0 else semB
            pltpu.make_async_copy(z_in[r].at[srcv.at[0]], rows2[b], sems[0]).wait()
            pltpu.make_async_copy(el_in[r].at[srcv.at[0]], elg2[b], sems[1]).wait()
            pltpu.make_async_copy(er_in[r].at[dstv.at[0]], erg2[b], sems[2]).wait()

        def compute_scatter(r, ch, b):
            rows, elg, erg = rows2[b], elg2[b], erg2[b]

            @functools.partial(plsc.parallel_loop, 0, CH, unroll=8)
            def edge(i):
                t = elg[i, :] + erg[i, :]
                e = jnp.where(t >= 0.0, t, t * NEG_SLOPE)
                ex = jnp.exp(e)
                erg[i, :] = ex
                for h in range(H):
                    sv = _splat(ex, h)
                    for k2 in range(D // 16):
                        sl = pl.ds(h * D + k2 * 16, 16)
                        rows[i, sl] = rows[i, sl] * sv

            pltpu.sync_copy(rows, acc_sh.at[dstv.at[ch]], add=True)
            pltpu.sync_copy(erg, s_sh.at[dstv.at[ch]], add=True)

        for r in range(3):
            pltpu.sync_copy(src_h.at[wid, r], srcv)
            pltpu.sync_copy(dst_h.at[wid, r], dstv)
            def zissue(t, _):
                off = r0 + t * ZR
                pltpu.async_copy(zacc, acc_sh.at[pl.ds(off, ZR)], semZ[0])
                pltpu.async_copy(zs_v, s_sh.at[pl.ds(off, ZR)], semZ[1])
                return _
            lax.fori_loop(0, RPT // ZR, zissue, None)

            def zwait(t, _):
                off = r0 + t * ZR
                pltpu.make_async_copy(zacc, acc_sh.at[pl.ds(off, ZR)], semZ[0]).wait()
                pltpu.make_async_copy(zs_v, s_sh.at[pl.ds(off, ZR)], semZ[1]).wait()
                return _
            lax.fori_loop(0, RPT // ZR, zwait, None)
            plsc.subcore_barrier()

            issue(r, 0, 0)
            issue(r, 1, 1)

            def pair(cb, _):
                cA = cb * 2
                cB = cA + 1
                wait(r, 0)
                compute_scatter(r, cA, 0)

                @pl.when(cA + 2 < NCHUNK)
                def _():
                    issue(r, cA + 2, 0)

                @pl.when(cB < NCHUNK)
                def _():
                    wait(r, 1)
                    compute_scatter(r, cB, 1)

                    @pl.when(cB + 2 < NCHUNK)
                    def _():
                        issue(r, cB + 2, 1)
                return _
            lax.fori_loop(0, (NCHUNK + 1) // 2, pair, None)
            plsc.subcore_barrier()

            pltpu.sync_copy(acc_sh.at[pl.ds(r0, RPT)],
                            acc_out[r].at[cid, pl.ds(r0, RPT)])
            pltpu.sync_copy(s_sh.at[pl.ds(r0, RPT)],
                            s_out[r].at[cid, pl.ds(r0, RPT)])

    outs = k(zs[0], zs[1], zs[2], els[0], els[1], els[2],
             ers[0], ers[1], ers[2], src4, dst4)
    return outs[:3], outs[3:]


def _attn_mat(a):
    flat = a.reshape(HD)
    head = jnp.repeat(jnp.arange(H), D)
    return (head[:, None] == jnp.arange(16)[None, :]).astype(jnp.float32) * flat[:, None]


def _prep_edges(ei):
    src = ei[0].reshape(NW, ET)
    dst = ei[1].reshape(NW, ET)
    pad = jnp.full((NW, ETP - ET), N, jnp.int32)
    return (jnp.concatenate([src, pad], axis=1).reshape(NW, NCHUNK, CH),
            jnp.concatenate([dst, pad], axis=1).reshape(NW, NCHUNK, CH))


def _stack_edges(eis):
    srcs, dsts = zip(*[_prep_edges(ei) for ei in eis])
    return (jnp.stack(srcs, axis=1), jnp.stack(dsts, axis=1))


def kernel(x, ei0, ei1, ei2,
           l0_W0, l0_al0, l0_ar0, l0_b0, l0_W1, l0_al1, l0_ar1, l0_b1,
           l0_W2, l0_al2, l0_ar2, l0_b2,
           l1_W0, l1_al0, l1_ar0, l1_b0, l1_W1, l1_al1, l1_ar1, l1_b1,
           l1_W2, l1_al2, l1_ar2, l1_b2,
           lin_W, lin_b):
    f32 = jnp.float32
    x_pad = jnp.zeros((NP, IN_FEATS), f32).at[:N].set(x)

    W0 = jnp.stack([l0_W0, l0_W1, l0_W2])
    A0 = jnp.stack([_attn_mat(l0_al0), _attn_mat(l0_al1), _attn_mat(l0_al2)])
    B0 = jnp.stack([_attn_mat(l0_ar0), _attn_mat(l0_ar1), _attn_mat(l0_ar2)])
    bias0 = jnp.stack([l0_b0, l0_b1, l0_b2])
    W1 = jnp.stack([l1_W0, l1_W1, l1_W2])
    A1 = jnp.stack([_attn_mat(l1_al0), _attn_mat(l1_al1), _attn_mat(l1_al2)])
    B1 = jnp.stack([_attn_mat(l1_ar0), _attn_mat(l1_ar1), _attn_mat(l1_ar2)])
    bias1 = jnp.stack([l1_b0, l1_b1, l1_b2])

    src4, dst4 = _stack_edges((ei0, ei1, ei2))

    linW_pad = jnp.zeros((HD, 256), f32).at[:, :C].set(lin_W)
    linb_pad = jnp.zeros((256,), f32).at[:C].set(lin_b)

    z, el, er = _tc_dense(x_pad, W0, A0, B0)
    accs0, ss0 = _sc_layer((z[0], z[1], z[2]), (el[0], el[1], el[2]),
                           (er[0], er[1], er[2]), src4, dst4)
    z1, el1, er1 = _tc_combine_dense(accs0, ss0, bias0, W1, A1, B1)
    accs1, ss1 = _sc_layer((z1[0], z1[1], z1[2]), (el1[0], el1[1], el1[2]),
                           (er1[0], er1[1], er1[2]), src4, dst4)
    out = _tc_combine_linear(accs1, ss1, bias1, linW_pad, linb_pad)
    return out[:N, :C]


# speedup vs baseline: 1.0022x; 1.0022x over previous
import functools

import jax
import jax.numpy as jnp
from jax import lax
from jax.experimental import pallas as pl
from jax.experimental.pallas import tpu as pltpu
from jax.experimental.pallas import tpu_sc as plsc

N = 10000
E = 100000
IN_FEATS = 128
H = 4
D = 32
HD = H * D
C = 153
NEG_SLOPE = 0.2

NP = 10240
NC = 2
NS = 16
NW = NC * NS
ET = E // NW
CH = 64
NCHUNK = (ET + CH - 1) // CH
ETP = NCHUNK * CH
RPT = NP // NS
ZR = 8

_BLK = 1024


def _tc_dense(h, W3, A3, B3):
    def body(h_ref, w_ref, a_ref, b_ref, z_ref, el_ref, er_ref):
        hb = h_ref[...]
        for r in range(3):
            z = jnp.dot(hb, w_ref[r], preferred_element_type=jnp.float32)
            z_ref[r] = z
            el_ref[r] = jnp.dot(z, a_ref[r], preferred_element_type=jnp.float32)
            er_ref[r] = jnp.dot(z, b_ref[r], preferred_element_type=jnp.float32)

    return pl.pallas_call(
        body,
        grid=(NP // _BLK,),
        in_specs=[
            pl.BlockSpec((_BLK, IN_FEATS), lambda i: (i, 0)),
            pl.BlockSpec((3, IN_FEATS, HD), lambda i: (0, 0, 0)),
            pl.BlockSpec((3, HD, 16), lambda i: (0, 0, 0)),
            pl.BlockSpec((3, HD, 16), lambda i: (0, 0, 0)),
        ],
        out_specs=[
            pl.BlockSpec((3, _BLK, HD), lambda i: (0, i, 0)),
            pl.BlockSpec((3, _BLK, 16), lambda i: (0, i, 0)),
            pl.BlockSpec((3, _BLK, 16), lambda i: (0, i, 0)),
        ],
        out_shape=[
            jax.ShapeDtypeStruct((3, NP, HD), jnp.float32),
            jax.ShapeDtypeStruct((3, NP, 16), jnp.float32),
            jax.ShapeDtypeStruct((3, NP, 16), jnp.float32),
        ],
    )(h, W3, A3, B3)


def _head_expand_mat():
    col = lax.broadcasted_iota(jnp.int32, (H, HD), 1)
    row = lax.broadcasted_iota(jnp.int32, (H, HD), 0)
    return (col // D == row).astype(jnp.float32)


def _combine(acc_refs, s_refs, bias_ref):
    emat = _head_expand_mat()
    hsum = None
    for r in range(3):
        acc = acc_refs[r][0] + acc_refs[r][1]
        s = s_refs[r][0][:, :H] + s_refs[r][1][:, :H]
        inv = 1.0 / (s + 1e-9)
        sexp = jnp.dot(inv, emat, preferred_element_type=jnp.float32)
        t = acc * sexp + bias_ref[r][None, :]
        hsum = t if hsum is None else hsum + t
    return hsum


def _tc_combine_dense(accs, ss, bias3, W3, A3, B3):
    def body(a0, a1, a2, s0, s1, s2, bias_ref, w_ref, a3_ref, b3_ref,
             z_ref, el_ref, er_ref):
        h = jnp.maximum(_combine((a0, a1, a2), (s0, s1, s2), bias_ref), 0.0)
        for r in range(3):
            z = jnp.dot(h, w_ref[r], preferred_element_type=jnp.float32)
            z_ref[r] = z
            el_ref[r] = jnp.dot(z, a3_ref[r], preferred_element_type=jnp.float32)
            er_ref[r] = jnp.dot(z, b3_ref[r], preferred_element_type=jnp.float32)

    acc_spec = pl.BlockSpec((NC, _BLK, HD), lambda i: (0, i, 0))
    s_spec = pl.BlockSpec((NC, _BLK, 16), lambda i: (0, i, 0))
    return pl.pallas_call(
        body,
        grid=(NP // _BLK,),
        in_specs=[acc_spec, acc_spec, acc_spec, s_spec, s_spec, s_spec,
                  pl.BlockSpec((3, HD), lambda i: (0, 0)),
                  pl.BlockSpec((3, IN_FEATS, HD), lambda i: (0, 0, 0)),
                  pl.BlockSpec((3, HD, 16), lambda i: (0, 0, 0)),
                  pl.BlockSpec((3, HD, 16), lambda i: (0, 0, 0))],
        out_specs=[
            pl.BlockSpec((3, _BLK, HD), lambda i: (0, i, 0)),
            pl.BlockSpec((3, _BLK, 16), lambda i: (0, i, 0)),
            pl.BlockSpec((3, _BLK, 16), lambda i: (0, i, 0)),
        ],
        out_shape=[
            jax.ShapeDtypeStruct((3, NP, HD), jnp.float32),
            jax.ShapeDtypeStruct((3, NP, 16), jnp.float32),
            jax.ShapeDtypeStruct((3, NP, 16), jnp.float32),
        ],
    )(*accs, *ss, bias3, W3, A3, B3)


def _tc_combine_linear(accs, ss, bias3, linW_pad, linb_pad):
    def body(a0, a1, a2, s0, s1, s2, bias_ref, w_ref, b_ref, out_ref):
        h = _combine((a0, a1, a2), (s0, s1, s2), bias_ref)
        out_ref[...] = (jnp.dot(h, w_ref[...], preferred_element_type=jnp.float32)
                        + b_ref[...][None, :])

    acc_spec = pl.BlockSpec((NC, _BLK, HD), lambda i: (0, i, 0))
    s_spec = pl.BlockSpec((NC, _BLK, 16), lambda i: (0, i, 0))
    return pl.pallas_call(
        body,
        grid=(NP // _BLK,),
        in_specs=[acc_spec, acc_spec, acc_spec, s_spec, s_spec, s_spec,
                  pl.BlockSpec((3, HD), lambda i: (0, 0)),
                  pl.BlockSpec((HD, 256), lambda i: (0, 0)),
                  pl.BlockSpec((256,), lambda i: (0,))],
        out_specs=pl.BlockSpec((_BLK, 256), lambda i: (i, 0)),
        out_shape=jax.ShapeDtypeStruct((NP, 256), jnp.float32),
    )(*accs, *ss, bias3, linW_pad, linb_pad)


_SPLAT_DNUMS = lax.GatherDimensionNumbers(
    offset_dims=(), collapsed_slice_dims=(0,), start_index_map=(0,))


def _splat(v, lane):
    idx = jnp.full((16, 1), lane, jnp.int32)
    return lax.gather(v, idx, _SPLAT_DNUMS, (1,),
                      mode=lax.GatherScatterMode.PROMISE_IN_BOUNDS)


def _sc_layer(zs, els, ers, src4, dst4):
    mesh = plsc.VectorSubcoreMesh(core_axis_name="c", subcore_axis_name="s")
    out_type = ([jax.ShapeDtypeStruct((NC, NP, HD), jnp.float32) for _ in range(3)]
                + [jax.ShapeDtypeStruct((NC, NP, 16), jnp.float32) for _ in range(3)])

    @functools.partial(
        pl.kernel,
        out_type=out_type,
        mesh=mesh,
        compiler_params=pltpu.CompilerParams(use_tc_tiling_on_sc=False),
        scratch_types=[
            pltpu.VMEM_SHARED((NP, HD), jnp.float32),
            pltpu.VMEM_SHARED((NP, 16), jnp.float32),
            pltpu.VMEM((NCHUNK, CH), jnp.int32),
            pltpu.VMEM((NCHUNK, CH), jnp.int32),
            [pltpu.VMEM((CH, HD), jnp.float32) for _ in range(2)],
            [pltpu.VMEM((CH, 16), jnp.float32) for _ in range(2)],
            [pltpu.VMEM((CH, 16), jnp.float32) for _ in range(2)],
            [pltpu.VMEM((CH, 16), jnp.float32) for _ in range(2)],
            pltpu.VMEM((CH, HD), jnp.float32),
            pltpu.VMEM((ZR, HD), jnp.float32),
            pltpu.VMEM((ZR, 16), jnp.float32),
            [pltpu.SemaphoreType.DMA for _ in range(3)],
            [pltpu.SemaphoreType.DMA for _ in range(3)],
            [pltpu.SemaphoreType.DMA for _ in range(2)],
        ],
    )
    def k(z0, z1, z2, el0, el1, el2, er0, er1, er2, src_h, dst_h,
          acc0, acc1, acc2, sd0, sd1, sd2,
          acc_sh, s_sh, srcv, dstv, rows2, elg2, erg2, exb2, srows, zacc, zs_v,
          semA, semB, semZ):
        cid = lax.axis_index("c")
        sid = lax.axis_index("s")
        wid = cid * NS + sid
        r0 = sid * RPT

        zero16 = jnp.zeros((16,), jnp.float32)

        def zinit(i, _):
            for j in range(HD // 16):
                zacc[i, pl.ds(j * 16, 16)] = zero16
            zs_v[i, :] = zero16
            return _
        lax.fori_loop(0, ZR, zinit, None)

        z_in = (z0, z1, z2)
        el_in = (el0, el1, el2)
        er_in = (er0, er1, er2)
        acc_out = (acc0, acc1, acc2)
        s_out = (sd0, sd1, sd2)

        def issue(r, ch, b):
            pltpu.async_copy(z_in[r].at[srcv.at[ch]], rows2[b], semA[0] if b == 0 else semB[0])
            pltpu.async_copy(el_in[r].at[srcv.at[ch]], elg2[b], semA[1] if b == 0 else semB[1])
            pltpu.async_copy(er_in[r].at[dstv.at[ch]], erg2[b], semB[2] if b else semA[2])

        def wait(r, b):
            sems = semA if b == 0 else semB
            pltpu.make_async_copy(z_in[r].at[srcv.at[0]], rows2[b], sems[0]).wait()
            pltpu.make_async_copy(el_in[r].at[srcv.at[0]], elg2[b], sems[1]).wait()
            pltpu.make_async_copy(er_in[r].at[dstv.at[0]], erg2[b], sems[2]).wait()

        def compute_scatter(r, ch, b):
            rows, elg, erg, exb = rows2[b], elg2[b], erg2[b], exb2[b]

            @functools.partial(plsc.parallel_loop, 0, CH, unroll=8)
            def edge(i):
                t = elg[i, :] + erg[i, :]
                e = jnp.where(t >= 0.0, t, t * NEG_SLOPE)
                ex = jnp.exp(e)
                exb[i, :] = ex
                for h in range(H):
                    sv = _splat(ex, h)
                    for k2 in range(D // 16):
                        sl = pl.ds(h * D + k2 * 16, 16)
                        srows[i, sl] = rows[i, sl] * sv

            pltpu.sync_copy(srows, acc_sh.at[dstv.at[ch]], add=True)
            pltpu.sync_copy(exb, s_sh.at[dstv.at[ch]], add=True)

        for r in range(3):
            pltpu.sync_copy(src_h.at[wid, r], srcv)
            pltpu.sync_copy(dst_h.at[wid, r], dstv)
            def zissue(t, _):
                off = r0 + t * ZR
                pltpu.async_copy(zacc, acc_sh.at[pl.ds(off, ZR)], semZ[0])
                pltpu.async_copy(zs_v, s_sh.at[pl.ds(off, ZR)], semZ[1])
                return _
            lax.fori_loop(0, RPT // ZR, zissue, None)

            def zwait(t, _):
                off = r0 + t * ZR
                pltpu.make_async_copy(zacc, acc_sh.at[pl.ds(off, ZR)], semZ[0]).wait()
                pltpu.make_async_copy(zs_v, s_sh.at[pl.ds(off, ZR)], semZ[1]).wait()
                return _
            lax.fori_loop(0, RPT // ZR, zwait, None)
            plsc.subcore_barrier()

            issue(r, 0, 0)
            issue(r, 1, 1)

            def pair(cb, _):
                cA = cb * 2
                cB = cA + 1
                wait(r, 0)
                compute_scatter(r, cA, 0)

                @pl.when(cA + 2 < NCHUNK)
                def _():
                    issue(r, cA + 2, 0)

                @pl.when(cB < NCHUNK)
                def _():
                    wait(r, 1)
                    compute_scatter(r, cB, 1)

                    @pl.when(cB + 2 < NCHUNK)
                    def _():
                        issue(r, cB + 2, 1)
                return _
            lax.fori_loop(0, (NCHUNK + 1) // 2, pair, None)
            plsc.subcore_barrier()

            pltpu.sync_copy(acc_sh.at[pl.ds(r0, RPT)],
                            acc_out[r].at[cid, pl.ds(r0, RPT)])
            pltpu.sync_copy(s_sh.at[pl.ds(r0, RPT)],
                            s_out[r].at[cid, pl.ds(r0, RPT)])

    outs = k(zs[0], zs[1], zs[2], els[0], els[1], els[2],
             ers[0], ers[1], ers[2], src4, dst4)
    return outs[:3], outs[3:]


def _attn_mat(a):
    flat = a.reshape(HD)
    head = jnp.repeat(jnp.arange(H), D)
    return (head[:, None] == jnp.arange(16)[None, :]).astype(jnp.float32) * flat[:, None]


def _prep_edges(ei):
    src = ei[0].reshape(NW, ET)
    dst = ei[1].reshape(NW, ET)
    pad = jnp.full((NW, ETP - ET), N, jnp.int32)
    return (jnp.concatenate([src, pad], axis=1).reshape(NW, NCHUNK, CH),
            jnp.concatenate([dst, pad], axis=1).reshape(NW, NCHUNK, CH))


def _stack_edges(eis):
    srcs, dsts = zip(*[_prep_edges(ei) for ei in eis])
    return (jnp.stack(srcs, axis=1), jnp.stack(dsts, axis=1))


def kernel(x, ei0, ei1, ei2,
           l0_W0, l0_al0, l0_ar0, l0_b0, l0_W1, l0_al1, l0_ar1, l0_b1,
           l0_W2, l0_al2, l0_ar2, l0_b2,
           l1_W0, l1_al0, l1_ar0, l1_b0, l1_W1, l1_al1, l1_ar1, l1_b1,
           l1_W2, l1_al2, l1_ar2, l1_b2,
           lin_W, lin_b):
    f32 = jnp.float32
    x_pad = jnp.zeros((NP, IN_FEATS), f32).at[:N].set(x)

    W0 = jnp.stack([l0_W0, l0_W1, l0_W2])
    A0 = jnp.stack([_attn_mat(l0_al0), _attn_mat(l0_al1), _attn_mat(l0_al2)])
    B0 = jnp.stack([_attn_mat(l0_ar0), _attn_mat(l0_ar1), _attn_mat(l0_ar2)])
    bias0 = jnp.stack([l0_b0, l0_b1, l0_b2])
    W1 = jnp.stack([l1_W0, l1_W1, l1_W2])
    A1 = jnp.stack([_attn_mat(l1_al0), _attn_mat(l1_al1), _attn_mat(l1_al2)])
    B1 = jnp.stack([_attn_mat(l1_ar0), _attn_mat(l1_ar1), _attn_mat(l1_ar2)])
    bias1 = jnp.stack([l1_b0, l1_b1, l1_b2])

    src4, dst4 = _stack_edges((ei0, ei1, ei2))

    linW_pad = jnp.zeros((HD, 256), f32).at[:, :C].set(lin_W)
    linb_pad = jnp.zeros((256,), f32).at[:C].set(lin_b)

    z, el, er = _tc_dense(x_pad, W0, A0, B0)
    accs0, ss0 = _sc_layer((z[0], z[1], z[2]), (el[0], el[1], el[2]),
                           (er[0], er[1], er[2]), src4, dst4)
    z1, el1, er1 = _tc_combine_dense(accs0, ss0, bias0, W1, A1, B1)
    accs1, ss1 = _sc_layer((z1[0], z1[1], z1[2]), (el1[0], el1[1], el1[2]),
                           (er1[0], er1[1], er1[2]), src4, dst4)
    out = _tc_combine_linear(accs1, ss1, bias1, linW_pad, linb_pad)
    return out[:N, :C]
